# baseline (speedup 1.0000x reference)
import jax
import jax.numpy as jnp
from jax.experimental import pallas as pl


def _gcn2_kernel(x_ref, adj_ref, w1_ref, b1_ref, w2_ref, b2_ref, out_ref):
    adjv = adj_ref[...]
    n = adjv.shape[0]
    bt = x_ref.shape[0]

    dis_r = jax.lax.rsqrt(1.0 + jnp.sum(adjv, axis=0, keepdims=True))
    ones_c = jnp.ones((n, 1), dtype=jnp.float32)
    colsum_c = jax.lax.dot_general(
        adjv, ones_c, (((0,), (0,)), ((), ())),
        preferred_element_type=jnp.float32)
    dis_c = jax.lax.rsqrt(1.0 + colsum_c)

    r = jax.lax.broadcasted_iota(jnp.int32, (n, n), 0)
    c = jax.lax.broadcasted_iota(jnp.int32, (n, n), 1)
    ahat = adjv + jnp.where(r == c, 1.0, 0.0)
    a2 = ahat * dis_r
    a3 = a2 * dis_c

    w1t = w1_ref[...].T
    w2t = w2_ref[...].T
    b1 = b1_ref[...]
    b2 = b2_ref[...]
    f = w1_ref.shape[0]

    agg1 = jnp.concatenate(
        [
            jax.lax.dot_general(
                x_ref[b, :, :], a3, (((0,), (0,)), ((), ())),
                preferred_element_type=jnp.float32)
            for b in range(bt)
        ],
        axis=0,
    )

    h1 = jnp.concatenate(
        [
            jnp.maximum(
                jnp.dot(w1t, agg1[b * f:(b + 1) * f, :],
                        preferred_element_type=jnp.float32) + b1, 0.0)
            for b in range(bt)
        ],
        axis=0,
    )

    agg2 = jnp.dot(h1 * dis_r, a2, preferred_element_type=jnp.float32)

    h = w1_ref.shape[1]
    for b in range(bt):
        t2 = jnp.maximum(
            jnp.dot(w2t, agg2[b * h:(b + 1) * h, :],
                    preferred_element_type=jnp.float32) + b2, 0.0)
        out_ref[b, :, :] = t2.T


def kernel(x, adj, W1, b1, W2, b2):
    bt, n, _ = x.shape
    o = W2.shape[1]
    return pl.pallas_call(
        _gcn2_kernel,
        out_shape=jax.ShapeDtypeStruct((bt, n, o), jnp.float32),
    )(x, adj, W1, b1[:, None], W2, b2[:, None])


# speedup vs baseline: 1.4756x; 1.4756x over previous
import jax
import jax.numpy as jnp
from jax.experimental import pallas as pl


def _gcn2_kernel(x_ref, adj_ref, w1_ref, b1_ref, w2_ref, b2_ref, out_ref):
    adjv = adj_ref[...]
    n = adjv.shape[0]
    bt = x_ref.shape[0]
    f = w1_ref.shape[0]

    dis_r = jax.lax.rsqrt(1.0 + jnp.sum(adjv, axis=0, keepdims=True))
    ones_c = jnp.ones((n, 1), dtype=jnp.float32)
    colsum_c = jax.lax.dot_general(
        adjv, ones_c, (((0,), (0,)), ((), ())),
        preferred_element_type=jnp.float32)
    dis_c = jax.lax.rsqrt(1.0 + colsum_c)

    r = jax.lax.broadcasted_iota(jnp.int32, (n, n), 0)
    c = jax.lax.broadcasted_iota(jnp.int32, (n, n), 1)
    ahat = adjv + jnp.where(r == c, 1.0, 0.0)
    a2 = ahat * dis_r
    a3 = a2 * dis_c

    w1t = w1_ref[...].T
    w2t = w2_ref[...].T
    b1 = b1_ref[...]
    b2 = b2_ref[...]

    xr = x_ref[...].reshape(bt, n, f)

    agg1 = jnp.concatenate(
        [
            jax.lax.dot_general(
                xr[b], a3, (((0,), (0,)), ((), ())),
                preferred_element_type=jnp.float32)
            for b in range(bt)
        ],
        axis=0,
    )

    h1 = jnp.concatenate(
        [
            jnp.maximum(
                jnp.dot(w1t, agg1[b * f:(b + 1) * f, :],
                        preferred_element_type=jnp.float32) + b1, 0.0)
            for b in range(bt)
        ],
        axis=0,
    )

    agg2 = jnp.dot(h1 * dis_r, a2, preferred_element_type=jnp.float32)

    h = w1_ref.shape[1]
    out_ref[...] = jnp.concatenate(
        [
            jnp.maximum(
                jnp.dot(w2t, agg2[b * h:(b + 1) * h, :],
                        preferred_element_type=jnp.float32) + b2, 0.0)
            for b in range(bt)
        ],
        axis=0,
    )


def kernel(x, adj, W1, b1, W2, b2):
    bt, n, f = x.shape
    o = W2.shape[1]
    outp = pl.pallas_call(
        _gcn2_kernel,
        out_shape=jax.ShapeDtypeStruct((bt * o, n), jnp.float32),
    )(x.reshape(bt, n * f), adj, W1, b1[:, None], W2, b2[:, None])
    return outp.reshape(bt, o, n).transpose(0, 2, 1)


# speedup vs baseline: 3.0702x; 2.0806x over previous
import jax
import jax.numpy as jnp
from jax.experimental import pallas as pl


def _gcn2_kernel(xp_ref, adj_ref, w1_ref, b1_ref, w2_ref, b2_ref, out_ref):
    adjv = adj_ref[...]
    n = adjv.shape[0]
    dis = jax.lax.rsqrt(1.0 + jnp.sum(adjv, axis=0, keepdims=True))
    r = jax.lax.broadcasted_iota(jnp.int32, (n, n), 0)
    c = jax.lax.broadcasted_iota(jnp.int32, (n, n), 1)
    ahat = (adjv + jnp.where(r == c, 1.0, 0.0)) * dis

    w1t = w1_ref[...].T
    w2t = w2_ref[...].T
    b1 = b1_ref[...]
    b2 = b2_ref[...]
    f1 = w1t.shape[1]
    f2 = w2t.shape[1]
    nb = xp_ref.shape[0] // f1

    def layer(hcur, wt, f, bias):
        agg = jnp.dot(hcur * dis, ahat, preferred_element_type=jnp.float32)
        pieces = [
            jnp.maximum(
                jnp.dot(wt, agg[i * f:(i + 1) * f, :],
                        preferred_element_type=jnp.float32) + bias, 0.0)
            for i in range(nb)
        ]
        return jnp.concatenate(pieces, axis=0)

    h1 = layer(xp_ref[...], w1t, f1, b1)
    out_ref[...] = layer(h1, w2t, f2, b2)


def kernel(x, adj, W1, b1, W2, b2):
    bt, n, f = x.shape
    o = W2.shape[1]
    xp = x.transpose(0, 2, 1).reshape(bt * f, n)
    outp = pl.pallas_call(
        _gcn2_kernel,
        out_shape=jax.ShapeDtypeStruct((bt * o, n), jnp.float32),
    )(xp, adj, W1, b1[:, None], W2, b2[:, None])
    return outp.reshape(bt, o, n).transpose(0, 2, 1)
